# baseline (device time: 22134 ns/iter reference)
import jax
import jax.numpy as jnp
from jax import lax
from jax.experimental import pallas as pl
from jax.experimental.pallas import tpu as pltpu

N_DEV = 4
DH = 128
SCALE = 0.08838834764831843
BF16 = jnp.bfloat16
F32 = jnp.float32


def _attn_group(qg, k, v, sq):
    qs = jnp.concatenate(
        [qg[:, t * DH:(t + 1) * DH] for t in range(4)], axis=0)
    s = lax.dot_general(qs, k, (((1,), (1,)), ((), ())),
                        preferred_element_type=F32) * SCALE
    m = jnp.max(s, axis=1, keepdims=True)
    p = jnp.exp(s - m)
    l = jnp.sum(p, axis=1, keepdims=True)
    o = (jnp.dot(p.astype(BF16), v, preferred_element_type=F32) / l
         ).astype(BF16)
    return jnp.concatenate(
        [o[t * sq:(t + 1) * sq, :] for t in range(4)], axis=1)


def kernel(x, Wq, Wo, Wk, Wv):
    B, Sq, D = x.shape
    kv_cols = Wk.shape[1] // N_DEV
    Dout = Wo.shape[1]
    half = Dout // 2

    my_i = lax.axis_index("i")
    xb = x[0].astype(BF16)
    wq_b = Wq.astype(BF16)
    wo_b = Wo.astype(BF16)
    wk_b = lax.dynamic_slice_in_dim(Wk, my_i * kv_cols, kv_cols, 1).astype(BF16)
    wv_b = lax.dynamic_slice_in_dim(Wv, my_i * kv_cols, kv_cols, 1).astype(BF16)

    def body(x_ref, wq_ref, wo_ref, wk_ref, wv_ref, out_ref,
             s1send, s1recv, s2send, s2recv, send_sems, recv_sems):
        me = lax.axis_index("i")
        p1 = jnp.bitwise_xor(me, 1)
        p2 = 3 - me

        barrier_sem = pltpu.get_barrier_semaphore()
        for p in (p1, p2):
            pl.semaphore_signal(barrier_sem, inc=1, device_id=(p,),
                                device_id_type=pl.DeviceIdType.MESH)

        xv = x_ref[:]
        qa = jnp.dot(xv, wq_ref[:], preferred_element_type=F32).astype(BF16)
        kb = jnp.dot(xv, wk_ref[:], preferred_element_type=F32).astype(BF16)
        vb = jnp.dot(xv, wv_ref[:], preferred_element_type=F32).astype(BF16)
        attn0 = _attn_group(qa[:, :512], kb[:, :DH], vb[:, :DH], Sq)
        attn1 = _attn_group(qa[:, 512:], kb[:, DH:], vb[:, DH:], Sq)
        attn = jnp.concatenate([attn0, attn1], axis=1)

        pA = jnp.dot(attn, wo_ref[:, :half], preferred_element_type=F32)
        s1send[0] = pA.astype(BF16)
        pl.semaphore_wait(barrier_sem, 2)
        r1a = pltpu.make_async_remote_copy(
            src_ref=s1send.at[0], dst_ref=s1recv.at[0],
            send_sem=send_sems.at[0], recv_sem=recv_sems.at[0],
            device_id=(p1,), device_id_type=pl.DeviceIdType.MESH)
        r1a.start()

        pB = jnp.dot(attn, wo_ref[:, half:], preferred_element_type=F32)
        s1send[1] = pB.astype(BF16)
        r1b = pltpu.make_async_remote_copy(
            src_ref=s1send.at[1], dst_ref=s1recv.at[1],
            send_sem=send_sems.at[1], recv_sem=recv_sems.at[1],
            device_id=(p2,), device_id_type=pl.DeviceIdType.MESH)
        r1b.start()

        r1a.wait()
        accA = pA + s1recv[0].astype(F32)
        s2send[0] = accA.astype(BF16)
        r2a = pltpu.make_async_remote_copy(
            src_ref=s2send.at[0], dst_ref=s2recv.at[0],
            send_sem=send_sems.at[2], recv_sem=recv_sems.at[2],
            device_id=(p2,), device_id_type=pl.DeviceIdType.MESH)
        r2a.start()

        r1b.wait()
        accB = pB + s1recv[1].astype(F32)
        s2send[1] = accB.astype(BF16)
        r2b = pltpu.make_async_remote_copy(
            src_ref=s2send.at[1], dst_ref=s2recv.at[1],
            send_sem=send_sems.at[3], recv_sem=recv_sems.at[3],
            device_id=(p1,), device_id_type=pl.DeviceIdType.MESH)
        r2b.start()

        r2a.wait()
        out_ref[0, :, :half] = accA + s2recv[0].astype(F32)
        r2b.wait()
        out_ref[0, :, half:] = accB + s2recv[1].astype(F32)

    return pl.pallas_call(
        body,
        out_shape=jax.ShapeDtypeStruct((B, Sq, Dout), F32),
        in_specs=[pl.BlockSpec(memory_space=pltpu.VMEM)] * 5,
        out_specs=pl.BlockSpec(memory_space=pltpu.VMEM),
        scratch_shapes=[
            pltpu.VMEM((2, Sq, half), BF16),
            pltpu.VMEM((2, Sq, half), BF16),
            pltpu.VMEM((2, Sq, half), BF16),
            pltpu.VMEM((2, Sq, half), BF16),
            pltpu.SemaphoreType.DMA((4,)),
            pltpu.SemaphoreType.DMA((4,)),
        ],
        compiler_params=pltpu.CompilerParams(collective_id=0),
    )(xb, wq_b, wo_b, wk_b, wv_b)


# device time: 20449 ns/iter; 1.0824x vs baseline; 1.0824x over previous
import jax
import jax.numpy as jnp
from jax import lax
from jax.experimental import pallas as pl
from jax.experimental.pallas import tpu as pltpu

N_DEV = 4
DH = 128
NC = 2
SCALE = 0.08838834764831843
BF16 = jnp.bfloat16
F32 = jnp.float32


def _attn_group(qg, k, v, sq):
    qs = jnp.concatenate(
        [qg[:, t * DH:(t + 1) * DH] for t in range(4)], axis=0)
    s = lax.dot_general(qs, k, (((1,), (1,)), ((), ())),
                        preferred_element_type=F32) * SCALE
    m = jnp.max(s, axis=1, keepdims=True)
    p = jnp.exp(s - m)
    l = jnp.sum(p, axis=1, keepdims=True)
    o = (jnp.dot(p.astype(BF16), v, preferred_element_type=F32) / l
         ).astype(BF16)
    return jnp.concatenate(
        [o[t * sq:(t + 1) * sq, :] for t in range(4)], axis=1)


def kernel(x, Wq, Wo, Wk, Wv):
    B, Sq, D = x.shape
    kv_cols = Wk.shape[1] // N_DEV
    Dout = Wo.shape[1]
    half = Dout // 2
    rc = Sq // NC

    my_i = lax.axis_index("i")
    xb = x[0].astype(BF16)
    wq_b = Wq.astype(BF16)
    wo_b = Wo.astype(BF16)
    wk_b = lax.dynamic_slice_in_dim(Wk, my_i * kv_cols, kv_cols, 1).astype(BF16)
    wv_b = lax.dynamic_slice_in_dim(Wv, my_i * kv_cols, kv_cols, 1).astype(BF16)

    def body(x_ref, wq_ref, wo_ref, wk_ref, wv_ref, out_ref,
             s1send, s1recv, s2send, s2recv, send_sems, recv_sems):
        me = lax.axis_index("i")
        p1 = jnp.bitwise_xor(me, 1)
        p2 = 3 - me
        part = {0: p1, 1: p2}

        barrier_sem = pltpu.get_barrier_semaphore()
        for p in (p1, p2):
            pl.semaphore_signal(barrier_sem, inc=1, device_id=(p,),
                                device_id_type=pl.DeviceIdType.MESH)

        xv = x_ref[:]
        qa = jnp.dot(xv, wq_ref[:], preferred_element_type=F32).astype(BF16)
        kb = jnp.dot(xv, wk_ref[:], preferred_element_type=F32).astype(BF16)
        vb = jnp.dot(xv, wv_ref[:], preferred_element_type=F32).astype(BF16)

        def sem_idx(stage, h, r):
            return stage * 2 * NC + h * NC + r

        def rdma(stage, h, r, buf_s, buf_r, target):
            i = sem_idx(stage, h, r)
            return pltpu.make_async_remote_copy(
                src_ref=buf_s.at[h, r], dst_ref=buf_r.at[h, r],
                send_sem=send_sems.at[i], recv_sem=recv_sems.at[i],
                device_id=(target,), device_id_type=pl.DeviceIdType.MESH)

        barrier_done = False
        s1 = {}
        p_val = {}
        for r in range(NC):
            r0, r1 = r * rc, (r + 1) * rc
            attn_r = jnp.concatenate(
                [_attn_group(qa[r0:r1, :512], kb[:, :DH], vb[:, :DH], rc),
                 _attn_group(qa[r0:r1, 512:], kb[:, DH:], vb[:, DH:], rc)],
                axis=1)
            for h in range(2):
                p_hr = jnp.dot(attn_r, wo_ref[:, h * half:(h + 1) * half],
                               preferred_element_type=F32)
                p_val[h, r] = p_hr
                s1send[h, r] = p_hr.astype(BF16)
                if not barrier_done:
                    pl.semaphore_wait(barrier_sem, 2)
                    barrier_done = True
                s1[h, r] = rdma(0, h, r, s1send, s1recv, part[h])
                s1[h, r].start()

        s2 = {}
        acc = {}
        for r in range(NC):
            for h in range(2):
                s1[h, r].wait()
                a = p_val[h, r] + s1recv[h, r].astype(F32)
                acc[h, r] = a
                s2send[h, r] = a.astype(BF16)
                s2[h, r] = rdma(1, h, r, s2send, s2recv, part[1 - h])
                s2[h, r].start()

        for r in range(NC):
            for h in range(2):
                s2[h, r].wait()
                out_ref[0, r * rc:(r + 1) * rc, h * half:(h + 1) * half] = (
                    acc[h, r] + s2recv[h, r].astype(F32))

    return pl.pallas_call(
        body,
        out_shape=jax.ShapeDtypeStruct((B, Sq, Dout), F32),
        in_specs=[pl.BlockSpec(memory_space=pltpu.VMEM)] * 5,
        out_specs=pl.BlockSpec(memory_space=pltpu.VMEM),
        scratch_shapes=[
            pltpu.VMEM((2, NC, rc, half), BF16),
            pltpu.VMEM((2, NC, rc, half), BF16),
            pltpu.VMEM((2, NC, rc, half), BF16),
            pltpu.VMEM((2, NC, rc, half), BF16),
            pltpu.SemaphoreType.DMA((4 * NC,)),
            pltpu.SemaphoreType.DMA((4 * NC,)),
        ],
        compiler_params=pltpu.CompilerParams(collective_id=0),
    )(xb, wq_b, wo_b, wk_b, wv_b)
